# baseline (speedup 1.0000x reference)
import functools

import jax
import jax.numpy as jnp
import numpy as np
from jax import lax
from jax.experimental import pallas as pl
from jax.experimental.pallas import tpu as pltpu
from jax.experimental.pallas import tpu_sc as plsc

NUM_FIELDS = 26
BATCH = 16384
VOCAB = 1000
EMB_DIM = 128
SCALE = float(1.0 / np.sqrt(float(NUM_FIELDS)))

NC = 2
NS = 16
NW = NC * NS
B_PER_W = BATCH // NW
CHUNK = 128
NCHUNK = B_PER_W // CHUNK
NT = NUM_FIELDS * NCHUNK
NPAIR = NUM_FIELDS // 2
NGRP = EMB_DIM // 32
PACKED = EMB_DIM // 2
HIMASK = -65536


def _sc_combine(W_all, idx3d):
    mesh = plsc.VectorSubcoreMesh(core_axis_name="c", subcore_axis_name="s")

    @functools.partial(
        pl.kernel,
        mesh=mesh,
        out_type=jax.ShapeDtypeStruct((BATCH, EMB_DIM), jnp.float32),
        compiler_params=pltpu.CompilerParams(
            needs_layout_passes=False, use_tc_tiling_on_sc=False
        ),
        scratch_types=[
            pltpu.VMEM((NT, CHUNK), jnp.int32),
            pltpu.VMEM((CHUNK, PACKED), jnp.int32),
            pltpu.VMEM((CHUNK, PACKED), jnp.int32),
            pltpu.VMEM((CHUNK, PACKED), jnp.int32),
            pltpu.VMEM((CHUNK, PACKED), jnp.int32),
            pltpu.VMEM((B_PER_W, EMB_DIM), jnp.float32),
            pltpu.SemaphoreType.DMA,
            pltpu.SemaphoreType.DMA,
        ],
    )
    def body(W_hbm, idx_hbm, out_hbm, idx_v, a0, b0, a1, b1, acc_v, sem0, sem1):
        wid = lax.axis_index("s") * NC + lax.axis_index("c")
        base = wid * B_PER_W
        bufs = ((a0, b0), (a1, b1))
        sems = (sem0, sem1)

        pltpu.sync_copy(idx_hbm.at[wid], idx_v)

        def offbody(f, _):
            off = f * VOCAB
            for c in range(NCHUNK):
                row = c * NUM_FIELDS + f
                for j in range(CHUNK // 16):
                    sl = pl.ds(j * 16, 16)
                    idx_v[row, sl] = idx_v[row, sl] + off
            return 0

        lax.fori_loop(0, NUM_FIELDS, offbody, 0)

        def pstart(tbase, p, s):
            t = tbase + 2 * p
            pltpu.make_async_copy(W_hbm.at[idx_v.at[t]], bufs[s][0], sems[s]).start()
            pltpu.make_async_copy(W_hbm.at[idx_v.at[t + 1]], bufs[s][1], sems[s]).start()

        def pwait(tbase, p, s):
            t = tbase + 2 * p
            pltpu.make_async_copy(W_hbm.at[idx_v.at[t]], bufs[s][0], sems[s]).wait()
            pltpu.make_async_copy(W_hbm.at[idx_v.at[t + 1]], bufs[s][1], sems[s]).wait()

        def unpack2(buf_a, buf_b, r, k):
            xa = buf_a[r, pl.ds(k * 16, 16)]
            xb = buf_b[r, pl.ds(k * 16, 16)]
            lo = (plsc.bitcast(xa << 16, jnp.float32)
                  + plsc.bitcast(xb << 16, jnp.float32))
            hi = (plsc.bitcast(xa & HIMASK, jnp.float32)
                  + plsc.bitcast(xb & HIMASK, jnp.float32))
            return lo, hi

        def accum_pair(cbase, s, first):
            buf_a, buf_b = bufs[s]

            def accrow(r, _):
                for k in range(NGRP):
                    lo, hi = unpack2(buf_a, buf_b, r, k)
                    sl_lo = pl.ds(k * 32, 16)
                    sl_hi = pl.ds(k * 32 + 16, 16)
                    if first:
                        acc_v[cbase + r, sl_lo] = lo
                        acc_v[cbase + r, sl_hi] = hi
                    else:
                        plsc.addupdate(acc_v.at[cbase + r, sl_lo], lo)
                        plsc.addupdate(acc_v.at[cbase + r, sl_hi], hi)
                return 0

            lax.fori_loop(0, CHUNK, accrow, 0, unroll=2)

        for c in range(NCHUNK):
            cbase = c * CHUNK
            tbase = c * NUM_FIELDS

            pstart(tbase, 0, 0)
            pstart(tbase, 1, 1)
            pwait(tbase, 0, 0)
            accum_pair(cbase, 0, first=True)
            pstart(tbase, 2, 0)

            def pair_body(pp, _):
                for q in range(2):
                    p = 1 + pp * 2 + q
                    s = 1 - q
                    pwait(tbase, p, s)
                    accum_pair(cbase, s, first=False)

                    @pl.when(p + 2 < NPAIR)
                    def _():
                        pstart(tbase, p + 2, s)
                return 0

            lax.fori_loop(0, (NPAIR - 1) // 2, pair_body, 0)

        def scrow(r, _):
            for j in range(EMB_DIM // 16):
                sl = pl.ds(j * 16, 16)
                acc_v[r, sl] = acc_v[r, sl] * SCALE
            return 0

        lax.fori_loop(0, B_PER_W, scrow, 0, unroll=2)
        pltpu.sync_copy(acc_v, out_hbm.at[pl.ds(base, B_PER_W)])

    return body(W_all, idx3d)


def _column_perm():
    perm = np.empty((EMB_DIM,), np.int64)
    for k in range(NGRP):
        for j in range(16):
            perm[32 * k + 2 * j] = 32 * k + j
            perm[32 * k + 2 * j + 1] = 32 * k + 16 + j
    return perm


_PERM = _column_perm()


def kernel(idx_f0, W_f0, idx_f1, W_f1, idx_f2, W_f2, idx_f3, W_f3, idx_f4, W_f4, idx_f5, W_f5, idx_f6, W_f6, idx_f7, W_f7, idx_f8, W_f8, idx_f9, W_f9, idx_f10, W_f10, idx_f11, W_f11, idx_f12, W_f12, idx_f13, W_f13, idx_f14, W_f14, idx_f15, W_f15, idx_f16, W_f16, idx_f17, W_f17, idx_f18, W_f18, idx_f19, W_f19, idx_f20, W_f20, idx_f21, W_f21, idx_f22, W_f22, idx_f23, W_f23, idx_f24, W_f24, idx_f25, W_f25):
    fields = locals()
    Ws = [fields[f"W_f{i}"] for i in range(NUM_FIELDS)]
    idxs = [fields[f"idx_f{i}"] for i in range(NUM_FIELDS)]
    W_bf = jnp.concatenate(Ws, axis=0)[:, _PERM].astype(jnp.bfloat16)
    W_all = jax.lax.bitcast_convert_type(
        W_bf.reshape(NUM_FIELDS * VOCAB, PACKED, 2), jnp.int32
    )
    idx3d = (
        jnp.stack(idxs, axis=0)
        .astype(jnp.int32)
        .reshape(NUM_FIELDS, NW, NCHUNK, CHUNK)
        .transpose(1, 2, 0, 3)
        .reshape(NW, NT, CHUNK)
    )
    return _sc_combine(W_all, idx3d)


# speedup vs baseline: 2.2320x; 2.2320x over previous
import functools

import jax
import jax.numpy as jnp
import numpy as np
from jax import lax
from jax.experimental import pallas as pl
from jax.experimental.pallas import tpu as pltpu
from jax.experimental.pallas import tpu_sc as plsc

NUM_FIELDS = 26
BATCH = 16384
VOCAB = 1000
EMB_DIM = 128
SCALE = float(1.0 / np.sqrt(float(NUM_FIELDS)))

NC = 2
NS = 16
NW = NC * NS
B_PER_W = BATCH // NW
CHUNK = 128
NCHUNK = B_PER_W // CHUNK
NPAIR = NUM_FIELDS // 2
NGRP = EMB_DIM // 16


def _sc_combine(Ws, idxs):
    mesh = plsc.VectorSubcoreMesh(core_axis_name="c", subcore_axis_name="s")

    @functools.partial(
        pl.kernel,
        mesh=mesh,
        out_type=jax.ShapeDtypeStruct((BATCH, EMB_DIM), jnp.float32),
        compiler_params=pltpu.CompilerParams(
            needs_layout_passes=False, use_tc_tiling_on_sc=False
        ),
        scratch_types=[
            pltpu.VMEM((NUM_FIELDS * NCHUNK, CHUNK), jnp.int32),
            pltpu.VMEM((CHUNK, EMB_DIM), jnp.float32),
            pltpu.VMEM((CHUNK, EMB_DIM), jnp.float32),
            pltpu.VMEM((CHUNK, EMB_DIM), jnp.float32),
            pltpu.VMEM((CHUNK, EMB_DIM), jnp.float32),
            pltpu.VMEM((CHUNK, EMB_DIM), jnp.float32),
            pltpu.VMEM((CHUNK, EMB_DIM), jnp.float32),
            pltpu.SemaphoreType.DMA,
            pltpu.SemaphoreType.DMA,
            pltpu.SemaphoreType.DMA,
            pltpu.SemaphoreType.DMA,
            pltpu.SemaphoreType.DMA,
        ],
    )
    def body(*refs):
        W_hbm = refs[:NUM_FIELDS]
        idx_hbm = refs[NUM_FIELDS:2 * NUM_FIELDS]
        out_hbm = refs[2 * NUM_FIELDS]
        idx_v, a0, b0, a1, b1, acc0, acc1 = refs[2 * NUM_FIELDS + 1:2 * NUM_FIELDS + 8]
        sem0, sem1, semi, wb0, wb1 = refs[2 * NUM_FIELDS + 8:]
        wid = lax.axis_index("s") * NC + lax.axis_index("c")
        base = wid * B_PER_W
        bufs = ((a0, b0), (a1, b1))
        sems = (sem0, sem1)
        accs = (acc0, acc1)
        wbs = (wb0, wb1)

        for f in range(NUM_FIELDS):
            pltpu.make_async_copy(
                idx_hbm[f].at[pl.ds(wid * NCHUNK, NCHUNK)],
                idx_v.at[pl.ds(f * NCHUNK, NCHUNK)],
                semi,
            ).start()
        for f in range(NUM_FIELDS):
            pltpu.make_async_copy(
                idx_hbm[f].at[pl.ds(wid * NCHUNK, NCHUNK)],
                idx_v.at[pl.ds(f * NCHUNK, NCHUNK)],
                semi,
            ).wait()

        def pstart(c, p, s):
            fa, fb = 2 * p, 2 * p + 1
            pltpu.make_async_copy(
                W_hbm[fa].at[idx_v.at[fa * NCHUNK + c]], bufs[s][0], sems[s]
            ).start()
            pltpu.make_async_copy(
                W_hbm[fb].at[idx_v.at[fb * NCHUNK + c]], bufs[s][1], sems[s]
            ).start()

        def pwait(c, p, s):
            fa, fb = 2 * p, 2 * p + 1
            pltpu.make_async_copy(
                W_hbm[fa].at[idx_v.at[fa * NCHUNK + c]], bufs[s][0], sems[s]
            ).wait()
            pltpu.make_async_copy(
                W_hbm[fb].at[idx_v.at[fb * NCHUNK + c]], bufs[s][1], sems[s]
            ).wait()

        def accum_pair(acc_v, s, first):
            buf_a, buf_b = bufs[s]

            def accrow(r, _):
                for k in range(NGRP):
                    sl = pl.ds(k * 16, 16)
                    v = buf_a[r, sl] + buf_b[r, sl]
                    if first:
                        acc_v[r, sl] = v
                    else:
                        plsc.addupdate(acc_v.at[r, sl], v)
                return 0

            lax.fori_loop(0, CHUNK, accrow, 0, unroll=2)

        for c in range(NCHUNK):
            acc_v = accs[c % 2]

            pstart(c, 0, 0)
            pstart(c, 1, 1)
            pwait(c, 0, 0)
            if c >= 2:
                pltpu.make_async_copy(
                    acc_v, out_hbm.at[pl.ds(base + (c - 2) * CHUNK, CHUNK)], wbs[c % 2]
                ).wait()
            accum_pair(acc_v, 0, first=True)
            pstart(c, 2, 0)

            for p in range(1, NPAIR):
                s = p % 2
                pwait(c, p, s)
                accum_pair(acc_v, s, first=False)
                if p + 2 < NPAIR:
                    pstart(c, p + 2, s)

            def scrow(r, _):
                for k in range(NGRP):
                    sl = pl.ds(k * 16, 16)
                    acc_v[r, sl] = acc_v[r, sl] * SCALE
                return 0

            lax.fori_loop(0, CHUNK, scrow, 0, unroll=2)
            pltpu.make_async_copy(
                acc_v, out_hbm.at[pl.ds(base + c * CHUNK, CHUNK)], wbs[c % 2]
            ).start()

        for c in (NCHUNK - 2, NCHUNK - 1):
            pltpu.make_async_copy(
                accs[c % 2], out_hbm.at[pl.ds(base + c * CHUNK, CHUNK)], wbs[c % 2]
            ).wait()

    return body(*Ws, *idxs)


def kernel(idx_f0, W_f0, idx_f1, W_f1, idx_f2, W_f2, idx_f3, W_f3, idx_f4, W_f4, idx_f5, W_f5, idx_f6, W_f6, idx_f7, W_f7, idx_f8, W_f8, idx_f9, W_f9, idx_f10, W_f10, idx_f11, W_f11, idx_f12, W_f12, idx_f13, W_f13, idx_f14, W_f14, idx_f15, W_f15, idx_f16, W_f16, idx_f17, W_f17, idx_f18, W_f18, idx_f19, W_f19, idx_f20, W_f20, idx_f21, W_f21, idx_f22, W_f22, idx_f23, W_f23, idx_f24, W_f24, idx_f25, W_f25):
    fields = locals()
    Ws = [fields[f"W_f{i}"] for i in range(NUM_FIELDS)]
    idxs = [
        fields[f"idx_f{i}"].astype(jnp.int32).reshape(NW * NCHUNK, CHUNK)
        for i in range(NUM_FIELDS)
    ]
    return _sc_combine(Ws, idxs)
